# baseline (speedup 1.0000x reference)
import functools

import jax
import jax.numpy as jnp
from jax import lax
from jax.experimental import pallas as pl
from jax.experimental.pallas import tpu as pltpu
from jax.experimental.pallas import tpu_sc as plsc

_BINS = 32
_D = 128
_B = 2
_L = 512
_VOCAB = 2 * _BINS + 2

_NC = 2
_NS = 16
_NW = _NC * _NS
_ROWS_PER_W = (_B * _L) // _NW
_LANES = 16


def _body(res_hbm, table_hbm, out_hbm, r_v, splat_v, idx_v, row_v, sem):
    cid = lax.axis_index("c")
    sid = lax.axis_index("s")
    wid = sid * _NC + cid
    base_row = wid * _ROWS_PER_W
    bb = base_row // _L
    i0 = base_row - bb * _L

    pltpu.sync_copy(res_hbm.at[pl.ds(bb * _L, _L)], r_v)

    for c in range(_ROWS_PER_W):
        idx_v[c // 8, pl.ds((c % 8) * _LANES, _LANES)] = (
            jnp.zeros((_LANES,), jnp.int32) + (base_row + c)
        )
    sp_copies = [
        pltpu.make_async_copy(
            res_hbm.at[idx_v.at[q]], splat_v.at[pl.ds(q * 128, 128)], sem
        )
        for q in range(4)
    ]
    for c in sp_copies:
        c.start()
    for c in sp_copies:
        c.wait()

    def row_body(k, _):
        row = base_row + k
        r_i = splat_v[pl.ds(k * _LANES, _LANES)]
        for jj in range(_L // _LANES):
            chunk = r_v[pl.ds(jj * _LANES, _LANES)]
            d = jnp.clip(chunk - r_i, -_BINS, _BINS) + (_BINS + 1)
            idx_v[jj // 8, pl.ds((jj % 8) * _LANES, _LANES)] = d
        copies = [
            pltpu.make_async_copy(
                table_hbm.at[idx_v.at[q]],
                row_v.at[pl.ds(q * 128, 128)],
                sem,
            )
            for q in range(4)
        ]
        for c in copies:
            c.start()
        for c in copies:
            c.wait()
        pltpu.sync_copy(row_v, out_hbm.at[pl.ds(row * _L, _L)])
        return 0

    lax.fori_loop(0, _ROWS_PER_W, row_body, 0)


@functools.partial(jax.jit, static_argnames=())
def kernel(residue_index, table):
    res_flat = residue_index.reshape(_B * _L)
    mesh = plsc.VectorSubcoreMesh(core_axis_name="c", subcore_axis_name="s")
    out = pl.kernel(
        _body,
        out_type=jax.ShapeDtypeStruct((_B * _L * _L, _D), jnp.float32),
        mesh=mesh,
        scratch_types=[
            pltpu.VMEM((_L,), jnp.int32),
            pltpu.VMEM((_ROWS_PER_W * _LANES,), jnp.int32),
            pltpu.VMEM((4, 128), jnp.int32),
            pltpu.VMEM((_L, _D), jnp.float32),
            pltpu.SemaphoreType.DMA,
        ],
    )(res_flat, table)
    return out.reshape(_B, _L, _L, _D)
